# Initial kernel scaffold; baseline (speedup 1.0000x reference)
import jax
import jax.numpy as jnp
from jax.experimental import pallas as pl


def kernel(x, edge_index, W0, b0, g0, be0, m0, v0, W1, b1, g1, be1, m1, v1, W2, b2, Wm1, bm1, Wm2, bm2):
    raise NotImplementedError("write your pallas kernel here")


# speedup vs baseline: 19.7534x; 19.7534x over previous
import functools

import jax
import jax.numpy as jnp
from jax import lax
from jax.experimental import pallas as pl
from jax.experimental.pallas import tpu as pltpu
from jax.experimental.pallas import tpu_sc as plsc

N = 10000
D = 128
E = 320000
EPS = 1e-5

NC = 2
NS = 16
NW = NC * NS
B = 128
NBLK = -(-E // (NW * B))
EPW = NBLK * B
EPAD = EPW * NW
NPAD = 10240
RPT = NPAD // NS
GARBAGE = NPAD - N

_mesh = plsc.VectorSubcoreMesh(
    core_axis_name="c", subcore_axis_name="s", num_cores=NC, num_subcores=NS
)


@functools.partial(
    pl.kernel,
    out_type=jax.ShapeDtypeStruct((NC, NPAD, 16), jnp.float32),
    mesh=_mesh,
    scratch_types=[
        pltpu.VMEM((NBLK, B), jnp.int32),
        pltpu.VMEM((B, 16), jnp.float32),
        pltpu.VMEM((B, 16), jnp.float32),
        pltpu.VMEM_SHARED((NPAD, 16), jnp.float32),
    ],
)
def _deg_kernel(dst_hbm, degp_hbm, idx_v, ones_v, zb_v, deg_sh):
    c = lax.axis_index("c")
    s = lax.axis_index("s")
    wid = c * NS + s

    @pl.loop(0, B)
    def _fill(i):
        ones_v[i] = jnp.ones((16,), jnp.float32)
        zb_v[i] = jnp.zeros((16,), jnp.float32)

    base = s * RPT

    @pl.loop(0, RPT // B)
    def _zero(j):
        pltpu.sync_copy(zb_v, deg_sh.at[pl.ds(base + j * B, B)])

    plsc.subcore_barrier()
    pltpu.sync_copy(dst_hbm.at[wid], idx_v)

    @pl.loop(0, NBLK)
    def _acc(j):
        pltpu.sync_copy(ones_v, deg_sh.at[idx_v.at[j]], add=True)

    plsc.subcore_barrier()
    pltpu.sync_copy(deg_sh.at[pl.ds(base, RPT)], degp_hbm.at[c, pl.ds(base, RPT)])


@functools.partial(
    pl.kernel,
    out_type=jax.ShapeDtypeStruct((NC, NPAD, D), jnp.float32),
    mesh=_mesh,
    scratch_types=[
        pltpu.VMEM((NBLK, B), jnp.int32),
        pltpu.VMEM((NBLK, B), jnp.int32),
        pltpu.VMEM((B, D), jnp.float32),
        pltpu.VMEM_SHARED((NPAD, D), jnp.float32),
    ],
)
def _edge_kernel(hp_hbm, src_hbm, dst_hbm, accp_hbm, isrc_v, idst_v, rows_v, acc_sh):
    c = lax.axis_index("c")
    s = lax.axis_index("s")
    wid = c * NS + s

    @pl.loop(0, B)
    def _fill(i):
        for k in range(D // 16):
            rows_v[i, pl.ds(k * 16, 16)] = jnp.zeros((16,), jnp.float32)

    base = s * RPT

    @pl.loop(0, RPT // B)
    def _zero(j):
        pltpu.sync_copy(rows_v, acc_sh.at[pl.ds(base + j * B, B)])

    plsc.subcore_barrier()
    pltpu.sync_copy(src_hbm.at[wid], isrc_v)
    pltpu.sync_copy(dst_hbm.at[wid], idst_v)

    @pl.loop(0, NBLK)
    def _acc(j):
        pltpu.sync_copy(hp_hbm.at[isrc_v.at[j]], rows_v)
        pltpu.sync_copy(rows_v, acc_sh.at[idst_v.at[j]], add=True)

    plsc.subcore_barrier()
    pltpu.sync_copy(acc_sh.at[pl.ds(base, RPT)], accp_hbm.at[c, pl.ds(base, RPT)])


R = 2000
GRID = N // R


def _tc_first_body(degp_ref, x_ref, w_ref, dis_ref, hp_ref):
    deg = degp_ref[0, :, 0:1] + degp_ref[1, :, 0:1] + 1.0
    dis = lax.rsqrt(deg)
    dis_ref[...] = dis
    h = jnp.dot(x_ref[...], w_ref[...], preferred_element_type=jnp.float32)
    hp_ref[...] = h * dis


_tc_first = pl.pallas_call(
    _tc_first_body,
    grid=(GRID,),
    in_specs=[
        pl.BlockSpec((NC, R, 16), lambda i: (0, i, 0)),
        pl.BlockSpec((R, D), lambda i: (i, 0)),
        pl.BlockSpec((D, D), lambda i: (0, 0)),
    ],
    out_specs=[
        pl.BlockSpec((R, 1), lambda i: (i, 0)),
        pl.BlockSpec((R, D), lambda i: (i, 0)),
    ],
    out_shape=[
        jax.ShapeDtypeStruct((N, 1), jnp.float32),
        jax.ShapeDtypeStruct((N, D), jnp.float32),
    ],
)


def _tc_layer_body(accp_ref, hp_ref, dis_ref, b_ref, g_ref, be_ref, m_ref, v_ref,
                   w_ref, out_ref):
    dis = dis_ref[...]
    t = (accp_ref[0] + accp_ref[1] + hp_ref[...]) * dis + b_ref[...]
    t = (t - m_ref[...]) * lax.rsqrt(v_ref[...] + EPS) * g_ref[...] + be_ref[...]
    t = jnp.maximum(t, 0.0)
    out_ref[...] = jnp.dot(t, w_ref[...], preferred_element_type=jnp.float32) * dis


_vec = pl.BlockSpec((1, D), lambda i: (0, 0))
_mat = pl.BlockSpec((D, D), lambda i: (0, 0))
_rowblk = pl.BlockSpec((R, D), lambda i: (i, 0))
_accblk = pl.BlockSpec((NC, R, D), lambda i: (0, i, 0))
_disblk = pl.BlockSpec((R, 1), lambda i: (i, 0))

_tc_layer = pl.pallas_call(
    _tc_layer_body,
    grid=(GRID,),
    in_specs=[_accblk, _rowblk, _disblk, _vec, _vec, _vec, _vec, _vec, _mat],
    out_specs=_rowblk,
    out_shape=jax.ShapeDtypeStruct((N, D), jnp.float32),
)


def _tc_final_body(accp_ref, hp_ref, dis_ref, b_ref, wm1_ref, bm1_ref,
                   wm2_ref, bm2_ref, out_ref):
    t = (accp_ref[0] + accp_ref[1] + hp_ref[...]) * dis_ref[...] + b_ref[...]
    z = jnp.dot(t, wm1_ref[...], preferred_element_type=jnp.float32) + bm1_ref[...]
    z = jnp.maximum(z, 0.0)
    out_ref[...] = jnp.dot(z, wm2_ref[...], preferred_element_type=jnp.float32) + bm2_ref[...]


_tc_final = pl.pallas_call(
    _tc_final_body,
    grid=(GRID,),
    in_specs=[_accblk, _rowblk, _disblk, _vec, _mat, _vec, _mat, _vec],
    out_specs=_rowblk,
    out_shape=jax.ShapeDtypeStruct((N, D), jnp.float32),
)


def kernel(x, edge_index, W0, b0, g0, be0, m0, v0, W1, b1, g1, be1, m1, v1,
           W2, b2, Wm1, bm1, Wm2, bm2):
    src = edge_index[0].astype(jnp.int32)
    dst = edge_index[1].astype(jnp.int32)
    npad = EPAD - E
    ar = jnp.arange(npad, dtype=jnp.int32)
    src_p = jnp.concatenate([src, (ar * 131) % N]).reshape(NW, NBLK, B)
    dst_p = jnp.concatenate([dst, N + ar % GARBAGE]).reshape(NW, NBLK, B)

    r1 = lambda a: a.reshape(1, D)
    degp = _deg_kernel(dst_p)
    dis, h0p = _tc_first(degp, x, W0)
    acc0 = _edge_kernel(h0p, src_p, dst_p)
    h1p = _tc_layer(acc0, h0p, dis, r1(b0), r1(g0), r1(be0), r1(m0), r1(v0), W1)
    acc1 = _edge_kernel(h1p, src_p, dst_p)
    h2p = _tc_layer(acc1, h1p, dis, r1(b1), r1(g1), r1(be1), r1(m1), r1(v1), W2)
    acc2 = _edge_kernel(h2p, src_p, dst_p)
    return _tc_final(acc2, h2p, dis, r1(b2), Wm1, r1(bm1), Wm2, r1(bm2))


# speedup vs baseline: 24.1571x; 1.2229x over previous
import functools

import jax
import jax.numpy as jnp
from jax import lax
from jax.experimental import pallas as pl
from jax.experimental.pallas import tpu as pltpu
from jax.experimental.pallas import tpu_sc as plsc

N = 10000
D = 128
E = 320000
EPS = 1e-5

NC = 2
NS = 16
NW = NC * NS
B = 128
NBLK = 80
W = 16
NCHUNK = NBLK // W
EPW = NBLK * B
EPAD = EPW * NW
NPAD = 10240
RPT = NPAD // NS
GARBAGE = NPAD - N

_mesh = plsc.VectorSubcoreMesh(
    core_axis_name="c", subcore_axis_name="s", num_cores=NC, num_subcores=NS
)


@functools.partial(
    pl.kernel,
    out_type=jax.ShapeDtypeStruct((NC, NPAD, 16), jnp.float32),
    mesh=_mesh,
    scratch_types=[
        pltpu.VMEM((NBLK, B), jnp.int32),
        pltpu.VMEM((B, 16), jnp.float32),
        pltpu.VMEM((B, 16), jnp.float32),
        pltpu.VMEM_SHARED((NPAD, 16), jnp.float32),
    ],
)
def _deg_kernel(dst_hbm, degp_hbm, idx_v, ones_v, zb_v, deg_sh):
    c = lax.axis_index("c")
    s = lax.axis_index("s")
    wid = c * NS + s

    @pl.loop(0, B)
    def _fill(i):
        ones_v[i] = jnp.ones((16,), jnp.float32)
        zb_v[i] = jnp.zeros((16,), jnp.float32)

    base = s * RPT

    @pl.loop(0, RPT // B)
    def _zero(j):
        pltpu.sync_copy(zb_v, deg_sh.at[pl.ds(base + j * B, B)])

    plsc.subcore_barrier()
    pltpu.sync_copy(dst_hbm.at[wid], idx_v)

    @pl.loop(0, NBLK)
    def _acc(j):
        pltpu.sync_copy(ones_v, deg_sh.at[idx_v.at[j]], add=True)

    plsc.subcore_barrier()
    pltpu.sync_copy(deg_sh.at[pl.ds(base, RPT)], degp_hbm.at[c, pl.ds(base, RPT)])


@functools.partial(
    pl.kernel,
    out_type=jax.ShapeDtypeStruct((NC, NPAD, D), jnp.float32),
    mesh=_mesh,
    scratch_types=[
        pltpu.VMEM((W, B), jnp.int32),
        pltpu.VMEM((W, B), jnp.int32),
        pltpu.VMEM((B, D), jnp.float32),
        pltpu.VMEM((B, D), jnp.float32),
        pltpu.VMEM_SHARED((NPAD, D), jnp.float32),
        pltpu.SemaphoreType.DMA,
        pltpu.SemaphoreType.DMA,
        pltpu.SemaphoreType.DMA,
        pltpu.SemaphoreType.DMA,
    ],
)
def _edge_kernel(hp_hbm, src_hbm, dst_hbm, accp_hbm, isrc_v, idst_v,
                 rows0_v, rows1_v, acc_sh, gs0, gs1, ss0, ss1):
    c = lax.axis_index("c")
    s = lax.axis_index("s")
    wid = c * NS + s

    @pl.loop(0, B)
    def _fill(i):
        for k in range(D // 16):
            rows0_v[i, pl.ds(k * 16, 16)] = jnp.zeros((16,), jnp.float32)

    base = s * RPT

    @pl.loop(0, RPT // B)
    def _zero(j):
        pltpu.sync_copy(rows0_v, acc_sh.at[pl.ds(base + j * B, B)])

    plsc.subcore_barrier()

    def g_start(j, buf, sem):
        pltpu.async_copy(hp_hbm.at[isrc_v.at[j]], buf, sem)

    def g_wait(j, buf, sem):
        pltpu.make_async_copy(hp_hbm.at[isrc_v.at[j]], buf, sem).wait()

    def s_start(j, buf, sem):
        pltpu.async_copy(buf, acc_sh.at[idst_v.at[j]], sem, add=True)

    def s_wait(j, buf, sem):
        pltpu.make_async_copy(buf, acc_sh.at[idst_v.at[j]], sem).wait()

    @pl.loop(0, NCHUNK)
    def _chunk(c):
        pltpu.sync_copy(src_hbm.at[wid, pl.ds(c * W, W)], isrc_v)
        pltpu.sync_copy(dst_hbm.at[wid, pl.ds(c * W, W)], idst_v)
        g_start(0, rows0_v, gs0)
        for t in range(W // 2):
            j0, j1 = 2 * t, 2 * t + 1
            g_wait(j0, rows0_v, gs0)
            if t > 0:
                s_wait(j0 - 1, rows1_v, ss1)
            g_start(j1, rows1_v, gs1)
            s_start(j0, rows0_v, ss0)
            g_wait(j1, rows1_v, gs1)
            if t < W // 2 - 1:
                s_wait(j0, rows0_v, ss0)
                g_start(j0 + 2, rows0_v, gs0)
            s_start(j1, rows1_v, ss1)
        s_wait(W - 2, rows0_v, ss0)
        s_wait(W - 1, rows1_v, ss1)

    plsc.subcore_barrier()
    pltpu.sync_copy(acc_sh.at[pl.ds(base, RPT)], accp_hbm.at[c, pl.ds(base, RPT)])


R = 2000
GRID = N // R


def _tc_first_body(degp_ref, x_ref, w_ref, dis_ref, hp_ref):
    deg = degp_ref[0, :, 0:1] + degp_ref[1, :, 0:1] + 1.0
    dis = lax.rsqrt(deg)
    dis_ref[...] = dis
    h = jnp.dot(x_ref[...], w_ref[...], preferred_element_type=jnp.float32)
    hp_ref[...] = h * dis


_tc_first = pl.pallas_call(
    _tc_first_body,
    grid=(GRID,),
    in_specs=[
        pl.BlockSpec((NC, R, 16), lambda i: (0, i, 0)),
        pl.BlockSpec((R, D), lambda i: (i, 0)),
        pl.BlockSpec((D, D), lambda i: (0, 0)),
    ],
    out_specs=[
        pl.BlockSpec((R, 1), lambda i: (i, 0)),
        pl.BlockSpec((R, D), lambda i: (i, 0)),
    ],
    out_shape=[
        jax.ShapeDtypeStruct((N, 1), jnp.float32),
        jax.ShapeDtypeStruct((N, D), jnp.float32),
    ],
)


def _tc_layer_body(accp_ref, hp_ref, dis_ref, b_ref, g_ref, be_ref, m_ref, v_ref,
                   w_ref, out_ref):
    dis = dis_ref[...]
    t = (accp_ref[0] + accp_ref[1] + hp_ref[...]) * dis + b_ref[...]
    t = (t - m_ref[...]) * lax.rsqrt(v_ref[...] + EPS) * g_ref[...] + be_ref[...]
    t = jnp.maximum(t, 0.0)
    out_ref[...] = jnp.dot(t, w_ref[...], preferred_element_type=jnp.float32) * dis


_vec = pl.BlockSpec((1, D), lambda i: (0, 0))
_mat = pl.BlockSpec((D, D), lambda i: (0, 0))
_rowblk = pl.BlockSpec((R, D), lambda i: (i, 0))
_accblk = pl.BlockSpec((NC, R, D), lambda i: (0, i, 0))
_disblk = pl.BlockSpec((R, 1), lambda i: (i, 0))

_tc_layer = pl.pallas_call(
    _tc_layer_body,
    grid=(GRID,),
    in_specs=[_accblk, _rowblk, _disblk, _vec, _vec, _vec, _vec, _vec, _mat],
    out_specs=_rowblk,
    out_shape=jax.ShapeDtypeStruct((N, D), jnp.float32),
)


def _tc_final_body(accp_ref, hp_ref, dis_ref, b_ref, wm1_ref, bm1_ref,
                   wm2_ref, bm2_ref, out_ref):
    t = (accp_ref[0] + accp_ref[1] + hp_ref[...]) * dis_ref[...] + b_ref[...]
    z = jnp.dot(t, wm1_ref[...], preferred_element_type=jnp.float32) + bm1_ref[...]
    z = jnp.maximum(z, 0.0)
    out_ref[...] = jnp.dot(z, wm2_ref[...], preferred_element_type=jnp.float32) + bm2_ref[...]


_tc_final = pl.pallas_call(
    _tc_final_body,
    grid=(GRID,),
    in_specs=[_accblk, _rowblk, _disblk, _vec, _mat, _vec, _mat, _vec],
    out_specs=_rowblk,
    out_shape=jax.ShapeDtypeStruct((N, D), jnp.float32),
)


def kernel(x, edge_index, W0, b0, g0, be0, m0, v0, W1, b1, g1, be1, m1, v1,
           W2, b2, Wm1, bm1, Wm2, bm2):
    src = edge_index[0].astype(jnp.int32)
    dst = edge_index[1].astype(jnp.int32)
    npad = EPAD - E
    ar = jnp.arange(npad, dtype=jnp.int32)
    src_p = jnp.concatenate([src, (ar * 131) % N]).reshape(NW, NBLK, B)
    dst_p = jnp.concatenate([dst, N + ar % GARBAGE]).reshape(NW, NBLK, B)

    r1 = lambda a: a.reshape(1, D)
    degp = _deg_kernel(dst_p)
    dis, h0p = _tc_first(degp, x, W0)
    acc0 = _edge_kernel(h0p, src_p, dst_p)
    h1p = _tc_layer(acc0, h0p, dis, r1(b0), r1(g0), r1(be0), r1(m0), r1(v0), W1)
    acc1 = _edge_kernel(h1p, src_p, dst_p)
    h2p = _tc_layer(acc1, h1p, dis, r1(b1), r1(g1), r1(be1), r1(m1), r1(v1), W2)
    acc2 = _edge_kernel(h2p, src_p, dst_p)
    return _tc_final(acc2, h2p, dis, r1(b2), Wm1, r1(bm1), Wm2, r1(bm2))


# speedup vs baseline: 25.2111x; 1.0436x over previous
import functools

import jax
import jax.numpy as jnp
from jax import lax
from jax.experimental import pallas as pl
from jax.experimental.pallas import tpu as pltpu
from jax.experimental.pallas import tpu_sc as plsc

N = 10000
D = 128
E = 320000
EPS = 1e-5

NC = 2
NS = 16
NW = NC * NS
B = 128
NBLK = 80
W = 16
NCHUNK = NBLK // W
EPW = NBLK * B
EPAD = EPW * NW
NPAD = 10240
RPT = NPAD // NS
GARBAGE = NPAD - N

_mesh = plsc.VectorSubcoreMesh(
    core_axis_name="c", subcore_axis_name="s", num_cores=NC, num_subcores=NS
)


@functools.partial(
    pl.kernel,
    out_type=jax.ShapeDtypeStruct((NC, NPAD, 16), jnp.float32),
    mesh=_mesh,
    scratch_types=[
        pltpu.VMEM((NBLK, B), jnp.int32),
        pltpu.VMEM((B, 16), jnp.float32),
        pltpu.VMEM((B, 16), jnp.float32),
        pltpu.VMEM_SHARED((NPAD, 16), jnp.float32),
    ],
)
def _deg_kernel(dst_hbm, degp_hbm, idx_v, ones_v, zb_v, deg_sh):
    c = lax.axis_index("c")
    s = lax.axis_index("s")
    wid = c * NS + s

    @pl.loop(0, B)
    def _fill(i):
        ones_v[i] = jnp.ones((16,), jnp.float32)
        zb_v[i] = jnp.zeros((16,), jnp.float32)

    base = s * RPT

    @pl.loop(0, RPT // B)
    def _zero(j):
        pltpu.sync_copy(zb_v, deg_sh.at[pl.ds(base + j * B, B)])

    plsc.subcore_barrier()
    pltpu.sync_copy(dst_hbm.at[wid], idx_v)

    @pl.loop(0, NBLK)
    def _acc(j):
        pltpu.sync_copy(ones_v, deg_sh.at[idx_v.at[j]], add=True)

    plsc.subcore_barrier()
    pltpu.sync_copy(deg_sh.at[pl.ds(base, RPT)], degp_hbm.at[c, pl.ds(base, RPT)])


@functools.partial(
    pl.kernel,
    out_type=jax.ShapeDtypeStruct((NC, NPAD, D), jnp.float32),
    mesh=_mesh,
    scratch_types=[
        pltpu.VMEM((2, W, B), jnp.int32),
        pltpu.VMEM((2, W, B), jnp.int32),
        pltpu.VMEM((B, D), jnp.float32),
        pltpu.VMEM((B, D), jnp.float32),
        pltpu.VMEM_SHARED((NPAD, D), jnp.float32),
        pltpu.SemaphoreType.DMA,
        pltpu.SemaphoreType.DMA,
        pltpu.SemaphoreType.DMA,
        pltpu.SemaphoreType.DMA,
        pltpu.SemaphoreType.DMA,
    ],
)
def _edge_kernel(hp_hbm, src_hbm, dst_hbm, accp_hbm, isrc_v, idst_v,
                 rows0_v, rows1_v, acc_sh, gs0, gs1, ss0, ss1, xsem):
    c = lax.axis_index("c")
    s = lax.axis_index("s")
    wid = c * NS + s

    @pl.loop(0, B)
    def _fill(i):
        for k in range(D // 16):
            rows0_v[i, pl.ds(k * 16, 16)] = jnp.zeros((16,), jnp.float32)

    base = s * RPT

    @pl.loop(0, RPT // B)
    def _zero(j):
        pltpu.sync_copy(rows0_v, acc_sh.at[pl.ds(base + j * B, B)])

    plsc.subcore_barrier()

    def g_start(p, j, buf, sem):
        pltpu.async_copy(hp_hbm.at[isrc_v.at[p, j]], buf, sem)

    def g_wait(p, j, buf, sem):
        pltpu.make_async_copy(hp_hbm.at[isrc_v.at[p, j]], buf, sem).wait()

    def s_start(p, j, buf, sem):
        pltpu.async_copy(buf, acc_sh.at[idst_v.at[p, j]], sem, add=True)

    def s_wait(p, j, buf, sem):
        pltpu.make_async_copy(buf, acc_sh.at[idst_v.at[p, j]], sem).wait()

    def x_start(c, slot):
        pltpu.async_copy(src_hbm.at[wid, pl.ds(c * W, W)], isrc_v.at[slot], xsem)
        pltpu.async_copy(dst_hbm.at[wid, pl.ds(c * W, W)], idst_v.at[slot], xsem)

    def x_wait(c, slot):
        pltpu.make_async_copy(src_hbm.at[wid, pl.ds(c * W, W)], isrc_v.at[slot], xsem).wait()
        pltpu.make_async_copy(dst_hbm.at[wid, pl.ds(c * W, W)], idst_v.at[slot], xsem).wait()

    pltpu.sync_copy(src_hbm.at[wid, pl.ds(0, W)], isrc_v.at[0])
    pltpu.sync_copy(dst_hbm.at[wid, pl.ds(0, W)], idst_v.at[0])
    g_start(0, 0, rows0_v, gs0)

    @pl.loop(0, NCHUNK)
    def _chunk(c):
        p = lax.rem(c, 2)
        pn = 1 - p

        @pl.when(c < NCHUNK - 1)
        def _():
            x_start(c + 1, pn)

        for t in range(W // 2):
            j0, j1 = 2 * t, 2 * t + 1
            g_wait(p, j0, rows0_v, gs0)
            if t > 0:
                s_wait(p, j0 - 1, rows1_v, ss1)
            else:
                @pl.when(c > 0)
                def _():
                    s_wait(pn, W - 1, rows1_v, ss1)

            g_start(p, j1, rows1_v, gs1)
            s_start(p, j0, rows0_v, ss0)
            g_wait(p, j1, rows1_v, gs1)
            s_wait(p, j0, rows0_v, ss0)
            if t < W // 2 - 1:
                g_start(p, j0 + 2, rows0_v, gs0)
            else:
                @pl.when(c < NCHUNK - 1)
                def _():
                    x_wait(c + 1, pn)
                    g_start(pn, 0, rows0_v, gs0)

            s_start(p, j1, rows1_v, ss1)

    s_wait((NCHUNK - 1) % 2, W - 1, rows1_v, ss1)

    plsc.subcore_barrier()
    pltpu.sync_copy(acc_sh.at[pl.ds(base, RPT)], accp_hbm.at[c, pl.ds(base, RPT)])


R = 2000
GRID = N // R


def _tc_first_body(degp_ref, x_ref, w_ref, dis_ref, hp_ref):
    deg = degp_ref[0, :, 0:1] + degp_ref[1, :, 0:1] + 1.0
    dis = lax.rsqrt(deg)
    dis_ref[...] = dis
    h = jnp.dot(x_ref[...], w_ref[...], preferred_element_type=jnp.float32)
    hp_ref[...] = h * dis


_tc_first = pl.pallas_call(
    _tc_first_body,
    grid=(GRID,),
    in_specs=[
        pl.BlockSpec((NC, R, 16), lambda i: (0, i, 0)),
        pl.BlockSpec((R, D), lambda i: (i, 0)),
        pl.BlockSpec((D, D), lambda i: (0, 0)),
    ],
    out_specs=[
        pl.BlockSpec((R, 1), lambda i: (i, 0)),
        pl.BlockSpec((R, D), lambda i: (i, 0)),
    ],
    out_shape=[
        jax.ShapeDtypeStruct((N, 1), jnp.float32),
        jax.ShapeDtypeStruct((N, D), jnp.float32),
    ],
)


def _tc_layer_body(accp_ref, hp_ref, dis_ref, b_ref, g_ref, be_ref, m_ref, v_ref,
                   w_ref, out_ref):
    dis = dis_ref[...]
    t = (accp_ref[0] + accp_ref[1] + hp_ref[...]) * dis + b_ref[...]
    t = (t - m_ref[...]) * lax.rsqrt(v_ref[...] + EPS) * g_ref[...] + be_ref[...]
    t = jnp.maximum(t, 0.0)
    out_ref[...] = jnp.dot(t, w_ref[...], preferred_element_type=jnp.float32) * dis


_vec = pl.BlockSpec((1, D), lambda i: (0, 0))
_mat = pl.BlockSpec((D, D), lambda i: (0, 0))
_rowblk = pl.BlockSpec((R, D), lambda i: (i, 0))
_accblk = pl.BlockSpec((NC, R, D), lambda i: (0, i, 0))
_disblk = pl.BlockSpec((R, 1), lambda i: (i, 0))

_tc_layer = pl.pallas_call(
    _tc_layer_body,
    grid=(GRID,),
    in_specs=[_accblk, _rowblk, _disblk, _vec, _vec, _vec, _vec, _vec, _mat],
    out_specs=_rowblk,
    out_shape=jax.ShapeDtypeStruct((N, D), jnp.float32),
)


def _tc_final_body(accp_ref, hp_ref, dis_ref, b_ref, wm1_ref, bm1_ref,
                   wm2_ref, bm2_ref, out_ref):
    t = (accp_ref[0] + accp_ref[1] + hp_ref[...]) * dis_ref[...] + b_ref[...]
    z = jnp.dot(t, wm1_ref[...], preferred_element_type=jnp.float32) + bm1_ref[...]
    z = jnp.maximum(z, 0.0)
    out_ref[...] = jnp.dot(z, wm2_ref[...], preferred_element_type=jnp.float32) + bm2_ref[...]


_tc_final = pl.pallas_call(
    _tc_final_body,
    grid=(GRID,),
    in_specs=[_accblk, _rowblk, _disblk, _vec, _mat, _vec, _mat, _vec],
    out_specs=_rowblk,
    out_shape=jax.ShapeDtypeStruct((N, D), jnp.float32),
)


def kernel(x, edge_index, W0, b0, g0, be0, m0, v0, W1, b1, g1, be1, m1, v1,
           W2, b2, Wm1, bm1, Wm2, bm2):
    src = edge_index[0].astype(jnp.int32)
    dst = edge_index[1].astype(jnp.int32)
    npad = EPAD - E
    ar = jnp.arange(npad, dtype=jnp.int32)
    src_p = jnp.concatenate([src, (ar * 131) % N]).reshape(NW, NBLK, B)
    dst_p = jnp.concatenate([dst, N + ar % GARBAGE]).reshape(NW, NBLK, B)

    r1 = lambda a: a.reshape(1, D)
    degp = _deg_kernel(dst_p)
    dis, h0p = _tc_first(degp, x, W0)
    acc0 = _edge_kernel(h0p, src_p, dst_p)
    h1p = _tc_layer(acc0, h0p, dis, r1(b0), r1(g0), r1(be0), r1(m0), r1(v0), W1)
    acc1 = _edge_kernel(h1p, src_p, dst_p)
    h2p = _tc_layer(acc1, h1p, dis, r1(b1), r1(g1), r1(be1), r1(m1), r1(v1), W2)
    acc2 = _edge_kernel(h2p, src_p, dst_p)
    return _tc_final(acc2, h2p, dis, r1(b2), Wm1, r1(bm1), Wm2, r1(bm2))


# speedup vs baseline: 29.6526x; 1.1762x over previous
import functools

import jax
import jax.numpy as jnp
from jax import lax
from jax.experimental import pallas as pl
from jax.experimental.pallas import tpu as pltpu
from jax.experimental.pallas import tpu_sc as plsc

N = 10000
D = 128
E = 320000
EPS = 1e-5

NC = 2
NS = 16
NW = NC * NS
B = 128
NBLK = 80
W = 16
NCHUNK = NBLK // W
EPW = NBLK * B
EPAD = EPW * NW
NPAD = 10240
RPT = NPAD // NS
GARBAGE = NPAD - N

_mesh = plsc.VectorSubcoreMesh(
    core_axis_name="c", subcore_axis_name="s", num_cores=NC, num_subcores=NS
)


@functools.partial(
    pl.kernel,
    out_type=jax.ShapeDtypeStruct((NC, NPAD, 16), jnp.float32),
    mesh=_mesh,
    scratch_types=[
        pltpu.VMEM((NBLK, B), jnp.int32),
        pltpu.VMEM((B, 16), jnp.float32),
        pltpu.VMEM((B, 16), jnp.float32),
        pltpu.VMEM_SHARED((NPAD, 16), jnp.float32),
        pltpu.SemaphoreType.DMA,
    ],
)
def _deg_kernel(dst_hbm, degp_hbm, idx_v, ones_v, zb_v, deg_sh, dsem):
    c = lax.axis_index("c")
    s = lax.axis_index("s")
    wid = c * NS + s

    @pl.loop(0, B)
    def _fill(i):
        ones_v[i] = jnp.ones((16,), jnp.float32)
        zb_v[i] = jnp.zeros((16,), jnp.float32)

    base = s * RPT

    @pl.loop(0, RPT // B)
    def _zero(j):
        pltpu.sync_copy(zb_v, deg_sh.at[pl.ds(base + j * B, B)])

    plsc.subcore_barrier()
    pltpu.sync_copy(dst_hbm.at[wid], idx_v)

    DEPTH = 8

    @pl.loop(0, NBLK)
    def _acc(j):
        pltpu.async_copy(ones_v, deg_sh.at[idx_v.at[j]], dsem, add=True)

        @pl.when(j >= DEPTH - 1)
        def _():
            pltpu.make_async_copy(ones_v, deg_sh.at[idx_v.at[j]], dsem).wait()

    @pl.loop(0, DEPTH - 1)
    def _drain(j):
        pltpu.make_async_copy(ones_v, deg_sh.at[idx_v.at[j]], dsem).wait()

    plsc.subcore_barrier()
    pltpu.sync_copy(deg_sh.at[pl.ds(base, RPT)], degp_hbm.at[c, pl.ds(base, RPT)])


@functools.partial(
    pl.kernel,
    out_type=jax.ShapeDtypeStruct((NC, NPAD, D), jnp.float32),
    mesh=_mesh,
    scratch_types=[
        pltpu.VMEM((2, W, B), jnp.int32),
        pltpu.VMEM((2, W, B), jnp.int32),
        pltpu.VMEM((B, D), jnp.float32),
        pltpu.VMEM((B, D), jnp.float32),
        pltpu.VMEM_SHARED((NPAD, D), jnp.float32),
        pltpu.SemaphoreType.DMA,
        pltpu.SemaphoreType.DMA,
        pltpu.SemaphoreType.DMA,
        pltpu.SemaphoreType.DMA,
        pltpu.SemaphoreType.DMA,
    ],
)
def _edge_kernel(hp_hbm, src_hbm, dst_hbm, accp_hbm, isrc_v, idst_v,
                 rows0_v, rows1_v, acc_sh, gs0, gs1, ss0, ss1, xsem):
    c = lax.axis_index("c")
    s = lax.axis_index("s")
    wid = c * NS + s

    @pl.loop(0, B)
    def _fill(i):
        for k in range(D // 16):
            rows0_v[i, pl.ds(k * 16, 16)] = jnp.zeros((16,), jnp.float32)

    base = s * RPT

    @pl.loop(0, RPT // B)
    def _zero(j):
        pltpu.sync_copy(rows0_v, acc_sh.at[pl.ds(base + j * B, B)])

    plsc.subcore_barrier()

    def g_start(p, j, buf, sem):
        pltpu.async_copy(hp_hbm.at[isrc_v.at[p, j]], buf, sem)

    def g_wait(p, j, buf, sem):
        pltpu.make_async_copy(hp_hbm.at[isrc_v.at[p, j]], buf, sem).wait()

    def s_start(p, j, buf, sem):
        pltpu.async_copy(buf, acc_sh.at[idst_v.at[p, j]], sem, add=True)

    def s_wait(p, j, buf, sem):
        pltpu.make_async_copy(buf, acc_sh.at[idst_v.at[p, j]], sem).wait()

    def x_start(c, slot):
        pltpu.async_copy(src_hbm.at[wid, pl.ds(c * W, W)], isrc_v.at[slot], xsem)
        pltpu.async_copy(dst_hbm.at[wid, pl.ds(c * W, W)], idst_v.at[slot], xsem)

    def x_wait(c, slot):
        pltpu.make_async_copy(src_hbm.at[wid, pl.ds(c * W, W)], isrc_v.at[slot], xsem).wait()
        pltpu.make_async_copy(dst_hbm.at[wid, pl.ds(c * W, W)], idst_v.at[slot], xsem).wait()

    pltpu.sync_copy(src_hbm.at[wid, pl.ds(0, W)], isrc_v.at[0])
    pltpu.sync_copy(dst_hbm.at[wid, pl.ds(0, W)], idst_v.at[0])
    g_start(0, 0, rows0_v, gs0)

    @pl.loop(0, NCHUNK)
    def _chunk(c):
        p = lax.rem(c, 2)
        pn = 1 - p

        @pl.when(c < NCHUNK - 1)
        def _():
            x_start(c + 1, pn)

        for t in range(W // 2):
            j0, j1 = 2 * t, 2 * t + 1
            if t > 0:
                s_wait(p, j0 - 1, rows1_v, ss1)
            else:
                @pl.when(c > 0)
                def _():
                    s_wait(pn, W - 1, rows1_v, ss1)

            g_start(p, j1, rows1_v, gs1)
            g_wait(p, j0, rows0_v, gs0)
            s_start(p, j0, rows0_v, ss0)
            s_wait(p, j0, rows0_v, ss0)
            if t < W // 2 - 1:
                g_start(p, j0 + 2, rows0_v, gs0)
            else:
                @pl.when(c < NCHUNK - 1)
                def _():
                    x_wait(c + 1, pn)
                    g_start(pn, 0, rows0_v, gs0)

            g_wait(p, j1, rows1_v, gs1)
            s_start(p, j1, rows1_v, ss1)

    s_wait((NCHUNK - 1) % 2, W - 1, rows1_v, ss1)

    plsc.subcore_barrier()
    pltpu.sync_copy(acc_sh.at[pl.ds(base, RPT)], accp_hbm.at[c, pl.ds(base, RPT)])


R = 2000
GRID = N // R


def _tc_first_body(degp_ref, x_ref, w_ref, dis_ref, hp_ref):
    deg = degp_ref[0, :, 0:1] + degp_ref[1, :, 0:1] + 1.0
    dis = lax.rsqrt(deg)
    dis_ref[...] = dis
    h = jnp.dot(x_ref[...], w_ref[...], preferred_element_type=jnp.float32)
    hp_ref[...] = h * dis


_tc_first = pl.pallas_call(
    _tc_first_body,
    grid=(GRID,),
    in_specs=[
        pl.BlockSpec((NC, R, 16), lambda i: (0, i, 0)),
        pl.BlockSpec((R, D), lambda i: (i, 0)),
        pl.BlockSpec((D, D), lambda i: (0, 0)),
    ],
    out_specs=[
        pl.BlockSpec((R, 1), lambda i: (i, 0)),
        pl.BlockSpec((R, D), lambda i: (i, 0)),
    ],
    out_shape=[
        jax.ShapeDtypeStruct((N, 1), jnp.float32),
        jax.ShapeDtypeStruct((N, D), jnp.float32),
    ],
)


def _tc_layer_body(accp_ref, hp_ref, dis_ref, b_ref, g_ref, be_ref, m_ref, v_ref,
                   w_ref, out_ref):
    dis = dis_ref[...]
    t = (accp_ref[0] + accp_ref[1] + hp_ref[...]) * dis + b_ref[...]
    t = (t - m_ref[...]) * lax.rsqrt(v_ref[...] + EPS) * g_ref[...] + be_ref[...]
    t = jnp.maximum(t, 0.0)
    out_ref[...] = jnp.dot(t, w_ref[...], preferred_element_type=jnp.float32) * dis


_vec = pl.BlockSpec((1, D), lambda i: (0, 0))
_mat = pl.BlockSpec((D, D), lambda i: (0, 0))
_rowblk = pl.BlockSpec((R, D), lambda i: (i, 0))
_accblk = pl.BlockSpec((NC, R, D), lambda i: (0, i, 0))
_disblk = pl.BlockSpec((R, 1), lambda i: (i, 0))

_tc_layer = pl.pallas_call(
    _tc_layer_body,
    grid=(GRID,),
    in_specs=[_accblk, _rowblk, _disblk, _vec, _vec, _vec, _vec, _vec, _mat],
    out_specs=_rowblk,
    out_shape=jax.ShapeDtypeStruct((N, D), jnp.float32),
)


def _tc_final_body(accp_ref, hp_ref, dis_ref, b_ref, wm1_ref, bm1_ref,
                   wm2_ref, bm2_ref, out_ref):
    t = (accp_ref[0] + accp_ref[1] + hp_ref[...]) * dis_ref[...] + b_ref[...]
    z = jnp.dot(t, wm1_ref[...], preferred_element_type=jnp.float32) + bm1_ref[...]
    z = jnp.maximum(z, 0.0)
    out_ref[...] = jnp.dot(z, wm2_ref[...], preferred_element_type=jnp.float32) + bm2_ref[...]


_tc_final = pl.pallas_call(
    _tc_final_body,
    grid=(GRID,),
    in_specs=[_accblk, _rowblk, _disblk, _vec, _mat, _vec, _mat, _vec],
    out_specs=_rowblk,
    out_shape=jax.ShapeDtypeStruct((N, D), jnp.float32),
)


def kernel(x, edge_index, W0, b0, g0, be0, m0, v0, W1, b1, g1, be1, m1, v1,
           W2, b2, Wm1, bm1, Wm2, bm2):
    src = edge_index[0].astype(jnp.int32)
    dst = edge_index[1].astype(jnp.int32)
    npad = EPAD - E
    ar = jnp.arange(npad, dtype=jnp.int32)
    src_p = jnp.concatenate([src, (ar * 131) % N]).reshape(NW, NBLK, B)
    dst_p = jnp.concatenate([dst, N + ar % GARBAGE]).reshape(NW, NBLK, B)

    r1 = lambda a: a.reshape(1, D)
    degp = _deg_kernel(dst_p)
    dis, h0p = _tc_first(degp, x, W0)
    acc0 = _edge_kernel(h0p, src_p, dst_p)
    h1p = _tc_layer(acc0, h0p, dis, r1(b0), r1(g0), r1(be0), r1(m0), r1(v0), W1)
    acc1 = _edge_kernel(h1p, src_p, dst_p)
    h2p = _tc_layer(acc1, h1p, dis, r1(b1), r1(g1), r1(be1), r1(m1), r1(v1), W2)
    acc2 = _edge_kernel(h2p, src_p, dst_p)
    return _tc_final(acc2, h2p, dis, r1(b2), Wm1, r1(bm1), Wm2, r1(bm2))


# speedup vs baseline: 29.8589x; 1.0070x over previous
import functools

import jax
import jax.numpy as jnp
from jax import lax
from jax.experimental import pallas as pl
from jax.experimental.pallas import tpu as pltpu
from jax.experimental.pallas import tpu_sc as plsc

N = 10000
D = 128
E = 320000
EPS = 1e-5

NC = 2
NS = 16
NW = NC * NS
B = 128
NBLK = 80
W = 16
NCHUNK = NBLK // W
EPW = NBLK * B
EPAD = EPW * NW
NPAD = 10240
RPT = NPAD // NS
GARBAGE = NPAD - N

_mesh = plsc.VectorSubcoreMesh(
    core_axis_name="c", subcore_axis_name="s", num_cores=NC, num_subcores=NS
)


@functools.partial(
    pl.kernel,
    out_type=jax.ShapeDtypeStruct((NC, NPAD, 16), jnp.float32),
    mesh=_mesh,
    scratch_types=[
        pltpu.VMEM((NBLK, B), jnp.int32),
        pltpu.VMEM((B, 16), jnp.float32),
        pltpu.VMEM((B, 16), jnp.float32),
        pltpu.VMEM_SHARED((NPAD, 16), jnp.float32),
        pltpu.SemaphoreType.DMA,
    ],
)
def _deg_kernel(dst_hbm, degp_hbm, idx_v, ones_v, zb_v, deg_sh, dsem):
    c = lax.axis_index("c")
    s = lax.axis_index("s")
    wid = c * NS + s

    @pl.loop(0, B)
    def _fill(i):
        ones_v[i] = jnp.ones((16,), jnp.float32)
        zb_v[i] = jnp.zeros((16,), jnp.float32)

    base = s * RPT

    @pl.loop(0, RPT // B)
    def _zero(j):
        pltpu.async_copy(zb_v, deg_sh.at[pl.ds(base + j * B, B)], dsem)

    @pl.loop(0, RPT // B)
    def _zdrain(j):
        pltpu.make_async_copy(zb_v, deg_sh.at[pl.ds(base + j * B, B)], dsem).wait()

    plsc.subcore_barrier()
    pltpu.sync_copy(dst_hbm.at[wid], idx_v)

    DEPTH = 8

    @pl.loop(0, NBLK)
    def _acc(j):
        pltpu.async_copy(ones_v, deg_sh.at[idx_v.at[j]], dsem, add=True)

        @pl.when(j >= DEPTH - 1)
        def _():
            pltpu.make_async_copy(ones_v, deg_sh.at[idx_v.at[j]], dsem).wait()

    @pl.loop(0, DEPTH - 1)
    def _drain(j):
        pltpu.make_async_copy(ones_v, deg_sh.at[idx_v.at[j]], dsem).wait()

    plsc.subcore_barrier()
    pltpu.sync_copy(deg_sh.at[pl.ds(base, RPT)], degp_hbm.at[c, pl.ds(base, RPT)])


@functools.partial(
    pl.kernel,
    out_type=jax.ShapeDtypeStruct((NC, NPAD, D), jnp.float32),
    mesh=_mesh,
    scratch_types=[
        pltpu.VMEM((2, W, B), jnp.int32),
        pltpu.VMEM((2, W, B), jnp.int32),
        pltpu.VMEM((B, D), jnp.float32),
        pltpu.VMEM((B, D), jnp.float32),
        pltpu.VMEM_SHARED((NPAD, D), jnp.float32),
        pltpu.SemaphoreType.DMA,
        pltpu.SemaphoreType.DMA,
        pltpu.SemaphoreType.DMA,
        pltpu.SemaphoreType.DMA,
        pltpu.SemaphoreType.DMA,
    ],
)
def _edge_kernel(hp_hbm, src_hbm, dst_hbm, accp_hbm, isrc_v, idst_v,
                 rows0_v, rows1_v, acc_sh, gs0, gs1, ss0, ss1, xsem):
    c = lax.axis_index("c")
    s = lax.axis_index("s")
    wid = c * NS + s

    @pl.loop(0, B)
    def _fill(i):
        for k in range(D // 16):
            rows0_v[i, pl.ds(k * 16, 16)] = jnp.zeros((16,), jnp.float32)

    base = s * RPT

    @pl.loop(0, RPT // B)
    def _zero(j):
        pltpu.async_copy(rows0_v, acc_sh.at[pl.ds(base + j * B, B)], xsem)

    @pl.loop(0, RPT // B)
    def _zdrain(j):
        pltpu.make_async_copy(rows0_v, acc_sh.at[pl.ds(base + j * B, B)], xsem).wait()

    plsc.subcore_barrier()

    def g_start(p, j, buf, sem):
        pltpu.async_copy(hp_hbm.at[isrc_v.at[p, j]], buf, sem)

    def g_wait(p, j, buf, sem):
        pltpu.make_async_copy(hp_hbm.at[isrc_v.at[p, j]], buf, sem).wait()

    def s_start(p, j, buf, sem):
        pltpu.async_copy(buf, acc_sh.at[idst_v.at[p, j]], sem, add=True)

    def s_wait(p, j, buf, sem):
        pltpu.make_async_copy(buf, acc_sh.at[idst_v.at[p, j]], sem).wait()

    def x_start(c, slot):
        pltpu.async_copy(src_hbm.at[wid, pl.ds(c * W, W)], isrc_v.at[slot], xsem)
        pltpu.async_copy(dst_hbm.at[wid, pl.ds(c * W, W)], idst_v.at[slot], xsem)

    def x_wait(c, slot):
        pltpu.make_async_copy(src_hbm.at[wid, pl.ds(c * W, W)], isrc_v.at[slot], xsem).wait()
        pltpu.make_async_copy(dst_hbm.at[wid, pl.ds(c * W, W)], idst_v.at[slot], xsem).wait()

    pltpu.sync_copy(src_hbm.at[wid, pl.ds(0, W)], isrc_v.at[0])
    pltpu.sync_copy(dst_hbm.at[wid, pl.ds(0, W)], idst_v.at[0])
    g_start(0, 0, rows0_v, gs0)

    @pl.loop(0, NCHUNK)
    def _chunk(c):
        p = lax.rem(c, 2)
        pn = 1 - p

        @pl.when(c < NCHUNK - 1)
        def _():
            x_start(c + 1, pn)

        for t in range(W // 2):
            j0, j1 = 2 * t, 2 * t + 1
            if t > 0:
                s_wait(p, j0 - 1, rows1_v, ss1)
            else:
                @pl.when(c > 0)
                def _():
                    s_wait(pn, W - 1, rows1_v, ss1)

            g_start(p, j1, rows1_v, gs1)
            g_wait(p, j0, rows0_v, gs0)
            s_start(p, j0, rows0_v, ss0)
            s_wait(p, j0, rows0_v, ss0)
            if t < W // 2 - 1:
                g_start(p, j0 + 2, rows0_v, gs0)
            else:
                @pl.when(c < NCHUNK - 1)
                def _():
                    x_wait(c + 1, pn)
                    g_start(pn, 0, rows0_v, gs0)

            g_wait(p, j1, rows1_v, gs1)
            s_start(p, j1, rows1_v, ss1)

    s_wait((NCHUNK - 1) % 2, W - 1, rows1_v, ss1)

    plsc.subcore_barrier()
    pltpu.sync_copy(acc_sh.at[pl.ds(base, RPT)], accp_hbm.at[c, pl.ds(base, RPT)])


R = 5000
GRID = N // R


def _tc_first_body(degp_ref, x_ref, w_ref, dis_ref, hp_ref):
    deg = degp_ref[0, :, 0:1] + degp_ref[1, :, 0:1] + 1.0
    dis = lax.rsqrt(deg)
    dis_ref[...] = dis
    h = jnp.dot(x_ref[...], w_ref[...], preferred_element_type=jnp.float32)
    hp_ref[...] = h * dis


_tc_first = pl.pallas_call(
    _tc_first_body,
    grid=(GRID,),
    in_specs=[
        pl.BlockSpec((NC, R, 16), lambda i: (0, i, 0)),
        pl.BlockSpec((R, D), lambda i: (i, 0)),
        pl.BlockSpec((D, D), lambda i: (0, 0)),
    ],
    out_specs=[
        pl.BlockSpec((R, 1), lambda i: (i, 0)),
        pl.BlockSpec((R, D), lambda i: (i, 0)),
    ],
    out_shape=[
        jax.ShapeDtypeStruct((N, 1), jnp.float32),
        jax.ShapeDtypeStruct((N, D), jnp.float32),
    ],
)


def _tc_layer_body(accp_ref, hp_ref, dis_ref, b_ref, g_ref, be_ref, m_ref, v_ref,
                   w_ref, out_ref):
    dis = dis_ref[...]
    t = (accp_ref[0] + accp_ref[1] + hp_ref[...]) * dis + b_ref[...]
    t = (t - m_ref[...]) * lax.rsqrt(v_ref[...] + EPS) * g_ref[...] + be_ref[...]
    t = jnp.maximum(t, 0.0)
    out_ref[...] = jnp.dot(t, w_ref[...], preferred_element_type=jnp.float32) * dis


_vec = pl.BlockSpec((1, D), lambda i: (0, 0))
_mat = pl.BlockSpec((D, D), lambda i: (0, 0))
_rowblk = pl.BlockSpec((R, D), lambda i: (i, 0))
_accblk = pl.BlockSpec((NC, R, D), lambda i: (0, i, 0))
_disblk = pl.BlockSpec((R, 1), lambda i: (i, 0))

_tc_layer = pl.pallas_call(
    _tc_layer_body,
    grid=(GRID,),
    in_specs=[_accblk, _rowblk, _disblk, _vec, _vec, _vec, _vec, _vec, _mat],
    out_specs=_rowblk,
    out_shape=jax.ShapeDtypeStruct((N, D), jnp.float32),
)


def _tc_final_body(accp_ref, hp_ref, dis_ref, b_ref, wm1_ref, bm1_ref,
                   wm2_ref, bm2_ref, out_ref):
    t = (accp_ref[0] + accp_ref[1] + hp_ref[...]) * dis_ref[...] + b_ref[...]
    z = jnp.dot(t, wm1_ref[...], preferred_element_type=jnp.float32) + bm1_ref[...]
    z = jnp.maximum(z, 0.0)
    out_ref[...] = jnp.dot(z, wm2_ref[...], preferred_element_type=jnp.float32) + bm2_ref[...]


_tc_final = pl.pallas_call(
    _tc_final_body,
    grid=(GRID,),
    in_specs=[_accblk, _rowblk, _disblk, _vec, _mat, _vec, _mat, _vec],
    out_specs=_rowblk,
    out_shape=jax.ShapeDtypeStruct((N, D), jnp.float32),
)


def kernel(x, edge_index, W0, b0, g0, be0, m0, v0, W1, b1, g1, be1, m1, v1,
           W2, b2, Wm1, bm1, Wm2, bm2):
    src = edge_index[0].astype(jnp.int32)
    dst = edge_index[1].astype(jnp.int32)
    npad = EPAD - E
    ar = jnp.arange(npad, dtype=jnp.int32)
    src_p = jnp.concatenate([src, (ar * 131) % N]).reshape(NW, NBLK, B)
    dst_p = jnp.concatenate([dst, N + ar % GARBAGE]).reshape(NW, NBLK, B)

    r1 = lambda a: a.reshape(1, D)
    degp = _deg_kernel(dst_p)
    dis, h0p = _tc_first(degp, x, W0)
    acc0 = _edge_kernel(h0p, src_p, dst_p)
    h1p = _tc_layer(acc0, h0p, dis, r1(b0), r1(g0), r1(be0), r1(m0), r1(v0), W1)
    acc1 = _edge_kernel(h1p, src_p, dst_p)
    h2p = _tc_layer(acc1, h1p, dis, r1(b1), r1(g1), r1(be1), r1(m1), r1(v1), W2)
    acc2 = _edge_kernel(h2p, src_p, dst_p)
    return _tc_final(acc2, h2p, dis, r1(b2), Wm1, r1(bm1), Wm2, r1(bm2))


# speedup vs baseline: 30.8339x; 1.0327x over previous
import functools

import jax
import jax.numpy as jnp
from jax import lax
from jax.experimental import pallas as pl
from jax.experimental.pallas import tpu as pltpu
from jax.experimental.pallas import tpu_sc as plsc

N = 10000
D = 128
E = 320000
EPS = 1e-5

NC = 2
NS = 16
NW = NC * NS
B = 80
NBLK = 128
W = 8
NCHUNK = NBLK // W
NBUF = 4
EPW = NBLK * B
EPAD = EPW * NW
NPAD = 10240
RPT = NPAD // NS
GARBAGE = NPAD - N

_mesh = plsc.VectorSubcoreMesh(
    core_axis_name="c", subcore_axis_name="s", num_cores=NC, num_subcores=NS
)


@functools.partial(
    pl.kernel,
    out_type=jax.ShapeDtypeStruct((NC, NPAD, 16), jnp.float32),
    mesh=_mesh,
    scratch_types=[
        pltpu.VMEM((NBLK, B), jnp.int32),
        pltpu.VMEM((B, 16), jnp.float32),
        pltpu.VMEM((B, 16), jnp.float32),
        pltpu.VMEM_SHARED((NPAD, 16), jnp.float32),
        pltpu.SemaphoreType.DMA,
    ],
)
def _deg_kernel(dst_hbm, degp_hbm, idx_v, ones_v, zb_v, deg_sh, dsem):
    c = lax.axis_index("c")
    s = lax.axis_index("s")
    wid = c * NS + s

    @pl.loop(0, B)
    def _fill(i):
        ones_v[i] = jnp.ones((16,), jnp.float32)
        zb_v[i] = jnp.zeros((16,), jnp.float32)

    base = s * RPT

    @pl.loop(0, RPT // B)
    def _zero(j):
        pltpu.async_copy(zb_v, deg_sh.at[pl.ds(base + j * B, B)], dsem)

    @pl.loop(0, RPT // B)
    def _zdrain(j):
        pltpu.make_async_copy(zb_v, deg_sh.at[pl.ds(base + j * B, B)], dsem).wait()

    plsc.subcore_barrier()
    pltpu.sync_copy(dst_hbm.at[wid], idx_v)

    DEPTH = 8

    @pl.loop(0, NBLK)
    def _acc(j):
        pltpu.async_copy(ones_v, deg_sh.at[idx_v.at[j]], dsem, add=True)

        @pl.when(j >= DEPTH - 1)
        def _():
            pltpu.make_async_copy(ones_v, deg_sh.at[idx_v.at[j]], dsem).wait()

    @pl.loop(0, DEPTH - 1)
    def _drain(j):
        pltpu.make_async_copy(ones_v, deg_sh.at[idx_v.at[j]], dsem).wait()

    plsc.subcore_barrier()
    pltpu.sync_copy(deg_sh.at[pl.ds(base, RPT)], degp_hbm.at[c, pl.ds(base, RPT)])


@functools.partial(
    pl.kernel,
    out_type=jax.ShapeDtypeStruct((NC, NPAD, D), jnp.float32),
    mesh=_mesh,
    scratch_types=[
        pltpu.VMEM((2, W, B), jnp.int32),
        pltpu.VMEM((2, W, B), jnp.int32),
        pltpu.VMEM((NBUF, B, D), jnp.float32),
        pltpu.VMEM_SHARED((NPAD, D), jnp.float32),
        [pltpu.SemaphoreType.DMA] * NBUF,
        [pltpu.SemaphoreType.DMA] * NBUF,
        pltpu.SemaphoreType.DMA,
    ],
)
def _edge_kernel(hp_hbm, src_hbm, dst_hbm, accp_hbm, isrc_v, idst_v,
                 rows_v, acc_sh, gs, ss, xsem):
    c = lax.axis_index("c")
    s = lax.axis_index("s")
    wid = c * NS + s

    @pl.loop(0, B)
    def _fill(i):
        for k in range(D // 16):
            rows_v[0, i, pl.ds(k * 16, 16)] = jnp.zeros((16,), jnp.float32)

    base = s * RPT

    @pl.loop(0, RPT // B)
    def _zero(j):
        pltpu.async_copy(rows_v.at[0], acc_sh.at[pl.ds(base + j * B, B)], xsem)

    @pl.loop(0, RPT // B)
    def _zdrain(j):
        pltpu.make_async_copy(rows_v.at[0], acc_sh.at[pl.ds(base + j * B, B)], xsem).wait()

    plsc.subcore_barrier()

    def g_start(p, j, b):
        pltpu.async_copy(hp_hbm.at[isrc_v.at[p, j]], rows_v.at[b], gs[b])

    def g_wait(p, j, b):
        pltpu.make_async_copy(hp_hbm.at[isrc_v.at[p, j]], rows_v.at[b], gs[b]).wait()

    def s_start(p, j, b):
        pltpu.async_copy(rows_v.at[b], acc_sh.at[idst_v.at[p, j]], ss[b], add=True)

    def s_wait(p, j, b):
        pltpu.make_async_copy(rows_v.at[b], acc_sh.at[idst_v.at[p, j]], ss[b]).wait()

    def x_start(c, slot):
        pltpu.async_copy(src_hbm.at[wid, pl.ds(c * W, W)], isrc_v.at[slot], xsem)
        pltpu.async_copy(dst_hbm.at[wid, pl.ds(c * W, W)], idst_v.at[slot], xsem)

    def x_wait(c, slot):
        pltpu.make_async_copy(src_hbm.at[wid, pl.ds(c * W, W)], isrc_v.at[slot], xsem).wait()
        pltpu.make_async_copy(dst_hbm.at[wid, pl.ds(c * W, W)], idst_v.at[slot], xsem).wait()

    pltpu.sync_copy(src_hbm.at[wid, pl.ds(0, W)], isrc_v.at[0])
    pltpu.sync_copy(dst_hbm.at[wid, pl.ds(0, W)], idst_v.at[0])
    g_start(0, 0, 0)
    g_start(0, 1, 1)
    g_start(0, 2, 2)

    @pl.loop(0, NCHUNK)
    def _chunk(c):
        p = lax.rem(c, 2)
        pn = 1 - p

        @pl.when(c < NCHUNK - 1)
        def _():
            x_start(c + 1, pn)

        for i in range(W):
            b = i % NBUF
            g_wait(p, i, b)
            s_start(p, i, b)
            if i > 0:
                s_wait(p, i - 1, (i - 1) % NBUF)
            else:
                @pl.when(c > 0)
                def _():
                    s_wait(pn, W - 1, (W - 1) % NBUF)

            if i < W - 3:
                g_start(p, i + 3, (i + 3) % NBUF)
            else:
                @pl.when(c < NCHUNK - 1)
                def _():
                    if i == W - 3:
                        x_wait(c + 1, pn)
                    g_start(pn, i + 3 - W, (i + 3) % NBUF)

    s_wait((NCHUNK - 1) % 2, W - 1, (W - 1) % NBUF)

    plsc.subcore_barrier()
    pltpu.sync_copy(acc_sh.at[pl.ds(base, RPT)], accp_hbm.at[c, pl.ds(base, RPT)])


R = 5000
GRID = N // R


def _tc_first_body(degp_ref, x_ref, w_ref, dis_ref, hp_ref):
    deg = degp_ref[0, :, 0:1] + degp_ref[1, :, 0:1] + 1.0
    dis = lax.rsqrt(deg)
    dis_ref[...] = dis
    h = jnp.dot(x_ref[...], w_ref[...], preferred_element_type=jnp.float32)
    hp_ref[...] = h * dis


_tc_first = pl.pallas_call(
    _tc_first_body,
    grid=(GRID,),
    in_specs=[
        pl.BlockSpec((NC, R, 16), lambda i: (0, i, 0)),
        pl.BlockSpec((R, D), lambda i: (i, 0)),
        pl.BlockSpec((D, D), lambda i: (0, 0)),
    ],
    out_specs=[
        pl.BlockSpec((R, 1), lambda i: (i, 0)),
        pl.BlockSpec((R, D), lambda i: (i, 0)),
    ],
    out_shape=[
        jax.ShapeDtypeStruct((N, 1), jnp.float32),
        jax.ShapeDtypeStruct((N, D), jnp.float32),
    ],
)


def _tc_layer_body(accp_ref, hp_ref, dis_ref, b_ref, g_ref, be_ref, m_ref, v_ref,
                   w_ref, out_ref):
    dis = dis_ref[...]
    t = (accp_ref[0] + accp_ref[1] + hp_ref[...]) * dis + b_ref[...]
    t = (t - m_ref[...]) * lax.rsqrt(v_ref[...] + EPS) * g_ref[...] + be_ref[...]
    t = jnp.maximum(t, 0.0)
    out_ref[...] = jnp.dot(t, w_ref[...], preferred_element_type=jnp.float32) * dis


_vec = pl.BlockSpec((1, D), lambda i: (0, 0))
_mat = pl.BlockSpec((D, D), lambda i: (0, 0))
_rowblk = pl.BlockSpec((R, D), lambda i: (i, 0))
_accblk = pl.BlockSpec((NC, R, D), lambda i: (0, i, 0))
_disblk = pl.BlockSpec((R, 1), lambda i: (i, 0))

_tc_layer = pl.pallas_call(
    _tc_layer_body,
    grid=(GRID,),
    in_specs=[_accblk, _rowblk, _disblk, _vec, _vec, _vec, _vec, _vec, _mat],
    out_specs=_rowblk,
    out_shape=jax.ShapeDtypeStruct((N, D), jnp.float32),
)


def _tc_final_body(accp_ref, hp_ref, dis_ref, b_ref, wm1_ref, bm1_ref,
                   wm2_ref, bm2_ref, out_ref):
    t = (accp_ref[0] + accp_ref[1] + hp_ref[...]) * dis_ref[...] + b_ref[...]
    z = jnp.dot(t, wm1_ref[...], preferred_element_type=jnp.float32) + bm1_ref[...]
    z = jnp.maximum(z, 0.0)
    out_ref[...] = jnp.dot(z, wm2_ref[...], preferred_element_type=jnp.float32) + bm2_ref[...]


_tc_final = pl.pallas_call(
    _tc_final_body,
    grid=(GRID,),
    in_specs=[_accblk, _rowblk, _disblk, _vec, _mat, _vec, _mat, _vec],
    out_specs=_rowblk,
    out_shape=jax.ShapeDtypeStruct((N, D), jnp.float32),
)


def kernel(x, edge_index, W0, b0, g0, be0, m0, v0, W1, b1, g1, be1, m1, v1,
           W2, b2, Wm1, bm1, Wm2, bm2):
    src = edge_index[0].astype(jnp.int32)
    dst = edge_index[1].astype(jnp.int32)
    npad = EPAD - E
    ar = jnp.arange(npad, dtype=jnp.int32)
    src_p = jnp.concatenate([src, (ar * 131) % N]).reshape(NW, NBLK, B)
    dst_p = jnp.concatenate([dst, N + ar % GARBAGE]).reshape(NW, NBLK, B)

    r1 = lambda a: a.reshape(1, D)
    degp = _deg_kernel(dst_p)
    dis, h0p = _tc_first(degp, x, W0)
    acc0 = _edge_kernel(h0p, src_p, dst_p)
    h1p = _tc_layer(acc0, h0p, dis, r1(b0), r1(g0), r1(be0), r1(m0), r1(v0), W1)
    acc1 = _edge_kernel(h1p, src_p, dst_p)
    h2p = _tc_layer(acc1, h1p, dis, r1(b1), r1(g1), r1(be1), r1(m1), r1(v1), W2)
    acc2 = _edge_kernel(h2p, src_p, dst_p)
    return _tc_final(acc2, h2p, dis, r1(b2), Wm1, r1(bm1), Wm2, r1(bm2))


# speedup vs baseline: 31.2215x; 1.0126x over previous
import functools

import jax
import jax.numpy as jnp
from jax import lax
from jax.experimental import pallas as pl
from jax.experimental.pallas import tpu as pltpu
from jax.experimental.pallas import tpu_sc as plsc

N = 10000
D = 128
E = 320000
EPS = 1e-5

NC = 2
NS = 16
NW = NC * NS
B = 80
NBLK = 128
W = 8
NCHUNK = NBLK // W
NBUF = 4
DB = 128
DNBLK = 80
EPW = NBLK * B
EPAD = EPW * NW
NPAD = 10240
RPT = NPAD // NS
GARBAGE = NPAD - N

_mesh = plsc.VectorSubcoreMesh(
    core_axis_name="c", subcore_axis_name="s", num_cores=NC, num_subcores=NS
)


@functools.partial(
    pl.kernel,
    out_type=jax.ShapeDtypeStruct((NC, NPAD, 16), jnp.float32),
    mesh=_mesh,
    scratch_types=[
        pltpu.VMEM((DNBLK, DB), jnp.int32),
        pltpu.VMEM((DB, 16), jnp.float32),
        pltpu.VMEM((DB, 16), jnp.float32),
        pltpu.VMEM_SHARED((NPAD, 16), jnp.float32),
        pltpu.SemaphoreType.DMA,
        pltpu.SemaphoreType.DMA,
    ],
)
def _deg_kernel(dst_hbm, degp_hbm, idx_v, ones_v, zb_v, deg_sh, dsem, zsem):
    c = lax.axis_index("c")
    s = lax.axis_index("s")
    wid = c * NS + s

    @pl.loop(0, DB)
    def _fill(i):
        ones_v[i] = jnp.ones((16,), jnp.float32)
        zb_v[i] = jnp.zeros((16,), jnp.float32)

    base = s * RPT

    @pl.loop(0, RPT // DB)
    def _zero(j):
        pltpu.async_copy(zb_v, deg_sh.at[pl.ds(base + j * DB, DB)], zsem)

    pltpu.sync_copy(dst_hbm.at[wid], idx_v)

    @pl.loop(0, RPT // DB)
    def _zdrain(j):
        pltpu.make_async_copy(zb_v, deg_sh.at[pl.ds(base + j * DB, DB)], zsem).wait()

    plsc.subcore_barrier()

    DEPTH = 16

    @pl.loop(0, DNBLK)
    def _acc(j):
        pltpu.async_copy(ones_v, deg_sh.at[idx_v.at[j]], dsem, add=True)

        @pl.when(j >= DEPTH - 1)
        def _():
            pltpu.make_async_copy(ones_v, deg_sh.at[idx_v.at[j]], dsem).wait()

    @pl.loop(0, DEPTH - 1)
    def _drain(j):
        pltpu.make_async_copy(ones_v, deg_sh.at[idx_v.at[j]], dsem).wait()

    plsc.subcore_barrier()
    pltpu.sync_copy(deg_sh.at[pl.ds(base, RPT)], degp_hbm.at[c, pl.ds(base, RPT)])


@functools.partial(
    pl.kernel,
    out_type=jax.ShapeDtypeStruct((NC, NPAD, D), jnp.float32),
    mesh=_mesh,
    scratch_types=[
        pltpu.VMEM((2, W, B), jnp.int32),
        pltpu.VMEM((2, W, B), jnp.int32),
        pltpu.VMEM((NBUF, B, D), jnp.float32),
        pltpu.VMEM_SHARED((NPAD, D), jnp.float32),
        [pltpu.SemaphoreType.DMA] * NBUF,
        [pltpu.SemaphoreType.DMA] * NBUF,
        pltpu.SemaphoreType.DMA,
    ],
)
def _edge_kernel(hp_hbm, src_hbm, dst_hbm, accp_hbm, isrc_v, idst_v,
                 rows_v, acc_sh, gs, ss, xsem):
    c = lax.axis_index("c")
    s = lax.axis_index("s")
    wid = c * NS + s

    @pl.loop(0, B)
    def _fill(i):
        for k in range(D // 16):
            rows_v[0, i, pl.ds(k * 16, 16)] = jnp.zeros((16,), jnp.float32)

    base = s * RPT

    @pl.loop(0, RPT // B)
    def _zero(j):
        pltpu.async_copy(rows_v.at[0], acc_sh.at[pl.ds(base + j * B, B)], xsem)

    def g_start(p, j, b):
        pltpu.async_copy(hp_hbm.at[isrc_v.at[p, j]], rows_v.at[b], gs[b])

    def g_wait(p, j, b):
        pltpu.make_async_copy(hp_hbm.at[isrc_v.at[p, j]], rows_v.at[b], gs[b]).wait()

    def s_start(p, j, b):
        pltpu.async_copy(rows_v.at[b], acc_sh.at[idst_v.at[p, j]], ss[b], add=True)

    def s_wait(p, j, b):
        pltpu.make_async_copy(rows_v.at[b], acc_sh.at[idst_v.at[p, j]], ss[b]).wait()

    def x_start(c, slot):
        pltpu.async_copy(src_hbm.at[wid, pl.ds(c * W, W)], isrc_v.at[slot], xsem)
        pltpu.async_copy(dst_hbm.at[wid, pl.ds(c * W, W)], idst_v.at[slot], xsem)

    def x_wait(c, slot):
        pltpu.make_async_copy(src_hbm.at[wid, pl.ds(c * W, W)], isrc_v.at[slot], xsem).wait()
        pltpu.make_async_copy(dst_hbm.at[wid, pl.ds(c * W, W)], idst_v.at[slot], xsem).wait()

    pltpu.sync_copy(src_hbm.at[wid, pl.ds(0, W)], isrc_v.at[0])
    pltpu.sync_copy(dst_hbm.at[wid, pl.ds(0, W)], idst_v.at[0])

    @pl.loop(0, RPT // B)
    def _zdrain(j):
        pltpu.make_async_copy(rows_v.at[0], acc_sh.at[pl.ds(base + j * B, B)], xsem).wait()

    g_start(0, 0, 0)
    g_start(0, 1, 1)
    g_start(0, 2, 2)
    plsc.subcore_barrier()

    @pl.loop(0, NCHUNK)
    def _chunk(c):
        p = lax.rem(c, 2)
        pn = 1 - p

        @pl.when(c < NCHUNK - 1)
        def _():
            x_start(c + 1, pn)

        for i in range(W):
            b = i % NBUF
            g_wait(p, i, b)
            s_start(p, i, b)
            if i > 0:
                s_wait(p, i - 1, (i - 1) % NBUF)
            else:
                @pl.when(c > 0)
                def _():
                    s_wait(pn, W - 1, (W - 1) % NBUF)

            if i < W - 3:
                g_start(p, i + 3, (i + 3) % NBUF)
            else:
                @pl.when(c < NCHUNK - 1)
                def _():
                    if i == W - 3:
                        x_wait(c + 1, pn)
                    g_start(pn, i + 3 - W, (i + 3) % NBUF)

    s_wait((NCHUNK - 1) % 2, W - 1, (W - 1) % NBUF)

    plsc.subcore_barrier()
    pltpu.sync_copy(acc_sh.at[pl.ds(base, RPT)], accp_hbm.at[c, pl.ds(base, RPT)])


R = 5000
GRID = N // R


def _tc_first_body(degp_ref, x_ref, w_ref, dis_ref, hp_ref):
    deg = degp_ref[0, :, 0:1] + degp_ref[1, :, 0:1] + 1.0
    dis = lax.rsqrt(deg)
    dis_ref[...] = dis
    h = jnp.dot(x_ref[...], w_ref[...], preferred_element_type=jnp.float32)
    hp_ref[...] = h * dis


_tc_first = pl.pallas_call(
    _tc_first_body,
    grid=(GRID,),
    in_specs=[
        pl.BlockSpec((NC, R, 16), lambda i: (0, i, 0)),
        pl.BlockSpec((R, D), lambda i: (i, 0)),
        pl.BlockSpec((D, D), lambda i: (0, 0)),
    ],
    out_specs=[
        pl.BlockSpec((R, 1), lambda i: (i, 0)),
        pl.BlockSpec((R, D), lambda i: (i, 0)),
    ],
    out_shape=[
        jax.ShapeDtypeStruct((N, 1), jnp.float32),
        jax.ShapeDtypeStruct((N, D), jnp.float32),
    ],
)


def _tc_layer_body(accp_ref, hp_ref, dis_ref, b_ref, g_ref, be_ref, m_ref, v_ref,
                   w_ref, out_ref):
    dis = dis_ref[...]
    t = (accp_ref[0] + accp_ref[1] + hp_ref[...]) * dis + b_ref[...]
    t = (t - m_ref[...]) * lax.rsqrt(v_ref[...] + EPS) * g_ref[...] + be_ref[...]
    t = jnp.maximum(t, 0.0)
    out_ref[...] = jnp.dot(t, w_ref[...], preferred_element_type=jnp.float32) * dis


_vec = pl.BlockSpec((1, D), lambda i: (0, 0))
_mat = pl.BlockSpec((D, D), lambda i: (0, 0))
_rowblk = pl.BlockSpec((R, D), lambda i: (i, 0))
_accblk = pl.BlockSpec((NC, R, D), lambda i: (0, i, 0))
_disblk = pl.BlockSpec((R, 1), lambda i: (i, 0))

_tc_layer = pl.pallas_call(
    _tc_layer_body,
    grid=(GRID,),
    in_specs=[_accblk, _rowblk, _disblk, _vec, _vec, _vec, _vec, _vec, _mat],
    out_specs=_rowblk,
    out_shape=jax.ShapeDtypeStruct((N, D), jnp.float32),
)


def _tc_final_body(accp_ref, hp_ref, dis_ref, b_ref, wm1_ref, bm1_ref,
                   wm2_ref, bm2_ref, out_ref):
    t = (accp_ref[0] + accp_ref[1] + hp_ref[...]) * dis_ref[...] + b_ref[...]
    z = jnp.dot(t, wm1_ref[...], preferred_element_type=jnp.float32) + bm1_ref[...]
    z = jnp.maximum(z, 0.0)
    out_ref[...] = jnp.dot(z, wm2_ref[...], preferred_element_type=jnp.float32) + bm2_ref[...]


_tc_final = pl.pallas_call(
    _tc_final_body,
    grid=(GRID,),
    in_specs=[_accblk, _rowblk, _disblk, _vec, _mat, _vec, _mat, _vec],
    out_specs=_rowblk,
    out_shape=jax.ShapeDtypeStruct((N, D), jnp.float32),
)


def kernel(x, edge_index, W0, b0, g0, be0, m0, v0, W1, b1, g1, be1, m1, v1,
           W2, b2, Wm1, bm1, Wm2, bm2):
    src = edge_index[0].astype(jnp.int32)
    dst = edge_index[1].astype(jnp.int32)
    npad = EPAD - E
    ar = jnp.arange(npad, dtype=jnp.int32)
    src_p = jnp.concatenate([src, (ar * 131) % N]).reshape(NW, NBLK, B)
    dst_pad = jnp.concatenate([dst, N + ar % GARBAGE])
    dst_p = dst_pad.reshape(NW, NBLK, B)
    dst_d = dst_pad.reshape(NW, DNBLK, DB)

    r1 = lambda a: a.reshape(1, D)
    degp = _deg_kernel(dst_d)
    dis, h0p = _tc_first(degp, x, W0)
    acc0 = _edge_kernel(h0p, src_p, dst_p)
    h1p = _tc_layer(acc0, h0p, dis, r1(b0), r1(g0), r1(be0), r1(m0), r1(v0), W1)
    acc1 = _edge_kernel(h1p, src_p, dst_p)
    h2p = _tc_layer(acc1, h1p, dis, r1(b1), r1(g1), r1(be1), r1(m1), r1(v1), W2)
    acc2 = _edge_kernel(h2p, src_p, dst_p)
    return _tc_final(acc2, h2p, dis, r1(b2), Wm1, r1(bm1), Wm2, r1(bm2))
